# Initial kernel scaffold; baseline (speedup 1.0000x reference)
import jax
import jax.numpy as jnp
from jax.experimental import pallas as pl


def kernel(inp_sess, mask_1, mask_inf, lengths, adj_items, item_emb, prob_emb, W_cls, W_ih, W_hh, b_ih, b_hh, a1, a2, ln1_g, ln1_b, ln2_g, ln2_b, ln3_g, ln3_b, ln4_g, ln4_b):
    raise NotImplementedError("write your pallas kernel here")


# speedup vs baseline: 1.3051x; 1.3051x over previous
import functools

import jax
import jax.numpy as jnp
from jax import lax
from jax.experimental import pallas as pl
from jax.experimental.pallas import tpu as pltpu
from jax.experimental.pallas import tpu_sc as plsc

HID = 128
NSAMP = 32
NNODE = 9999
NITEM = 10000
NP = 10240
ROUT_ITERS = 3
BLK_R = 128
BLK_V = 512
BQ = 2048
LSEQ = 50
GCHUNK = 128
NWORK = 32


def _sc_gather(table, idx):
    b = idx.shape[0]
    d = table.shape[1]
    per_w = b // NWORK
    nch = per_w // GCHUNK
    mesh = plsc.VectorSubcoreMesh(core_axis_name="c", subcore_axis_name="s")

    @functools.partial(
        pl.kernel,
        mesh=mesh,
        out_type=jax.ShapeDtypeStruct((b, d), jnp.float32),
        scratch_types=[
            pltpu.VMEM((GCHUNK,), jnp.int32),
            pltpu.VMEM((GCHUNK, d), jnp.float32),
            pltpu.SemaphoreType.DMA,
        ],
    )
    def gk(table_hbm, idx_hbm, out_hbm, idx_v, rows_v, sem):
        wid = lax.axis_index("s") * 2 + lax.axis_index("c")
        base = wid * per_w

        def body(i, carry):
            off = base + i * GCHUNK
            pltpu.sync_copy(idx_hbm.at[pl.ds(off, GCHUNK)], idx_v)
            pltpu.async_copy(table_hbm.at[idx_v], rows_v, sem).wait()
            pltpu.sync_copy(rows_v, out_hbm.at[pl.ds(off, GCHUNK)])
            return carry

        lax.fori_loop(0, nch, body, 0)

    return gk(table, idx)


def _l2n_kernel(x_ref, o_ref):
    x = x_ref[...]
    n = jnp.sqrt(jnp.sum(x * x, axis=1, keepdims=True))
    o_ref[...] = x / (n + 1e-12)


def _l2norm_rows(x):
    return pl.pallas_call(
        _l2n_kernel,
        grid=(NP // 512,),
        in_specs=[pl.BlockSpec((512, HID), lambda i: (i, 0))],
        out_specs=pl.BlockSpec((512, HID), lambda i: (i, 0)),
        out_shape=jax.ShapeDtypeStruct((NP, HID), jnp.float32),
    )(x)


def _routing_kernel(x_ref, u0_ref, nb_ref, o_ref):
    xx = x_ref[...]
    u = u0_ref[...]
    nb = nb_ref[...]
    for _ in range(ROUT_ITERS):
        logits = jnp.sum(nb * u[:, None, :], axis=2)
        m = jnp.max(logits, axis=1, keepdims=True)
        e = jnp.exp(logits - m)
        p = e / jnp.sum(e, axis=1, keepdims=True)
        v = xx + jnp.sum(p[:, :, None] * nb, axis=1)
        nrm = jnp.sqrt(jnp.sum(v * v, axis=1, keepdims=True))
        u = v / (nrm + 1e-12)
    o_ref[...] = u


def _routing(x, u0, nb):
    return pl.pallas_call(
        _routing_kernel,
        grid=(NP // BLK_R,),
        in_specs=[
            pl.BlockSpec((BLK_R, HID), lambda i: (i, 0)),
            pl.BlockSpec((BLK_R, HID), lambda i: (i, 0)),
            pl.BlockSpec((BLK_R, NSAMP, HID), lambda i: (i, 0, 0)),
        ],
        out_specs=pl.BlockSpec((BLK_R, HID), lambda i: (i, 0)),
        out_shape=jax.ShapeDtypeStruct((NP, HID), jnp.float32),
    )(x, u0, nb)


def _iv_kernel(x_ref, u1_ref, u2_ref, g_ref, b_ref, o_ref):
    i = pl.program_id(0)
    s = x_ref[...] + u1_ref[...] + u2_ref[...]
    m = jnp.mean(s, axis=1, keepdims=True)
    v = jnp.mean((s - m) ** 2, axis=1, keepdims=True)
    y = (s - m) / jnp.sqrt(v + 1e-5) * g_ref[...] + b_ref[...]
    rows = i * 512 + lax.broadcasted_iota(jnp.int32, y.shape, 0)
    o_ref[...] = jnp.where(rows < NNODE, y, 0.0)


def _iv(x, u1, u2, g, bb):
    return pl.pallas_call(
        _iv_kernel,
        grid=(NP // 512,),
        in_specs=[
            pl.BlockSpec((512, HID), lambda i: (i, 0)),
            pl.BlockSpec((512, HID), lambda i: (i, 0)),
            pl.BlockSpec((512, HID), lambda i: (i, 0)),
            pl.BlockSpec((1, HID), lambda i: (0, 0)),
            pl.BlockSpec((1, HID), lambda i: (0, 0)),
        ],
        out_specs=pl.BlockSpec((512, HID), lambda i: (i, 0)),
        out_shape=jax.ShapeDtypeStruct((NP, HID), jnp.float32),
    )(x, u1, u2, g, bb)


def _cls_kernel(pe_ref, wt_ref, g_ref, b_ref, o_ref):
    h = jnp.dot(pe_ref[...], wt_ref[...], preferred_element_type=jnp.float32)
    m = jnp.mean(h, axis=1, keepdims=True)
    v = jnp.mean((h - m) ** 2, axis=1, keepdims=True)
    o_ref[...] = (h - m) / jnp.sqrt(v + 1e-5) * g_ref[...] + b_ref[...]


def _cls_table(prob_emb, w_t, g, bb):
    return pl.pallas_call(
        _cls_kernel,
        out_shape=jax.ShapeDtypeStruct((NITEM, HID), jnp.float32),
    )(prob_emb, w_t, g, bb)


def _gru_kernel(x_ref, wih_ref, whh_ref, bih_ref, bhh_ref, len_ref,
                o_ref, h_ref):
    t = pl.program_id(0)

    @pl.when(t == 0)
    def _():
        h_ref[...] = jnp.zeros_like(h_ref)
        o_ref[...] = jnp.zeros_like(o_ref)

    xt = x_ref[0]
    h = h_ref[...]
    gi = jnp.dot(xt, wih_ref[...], preferred_element_type=jnp.float32) + bih_ref[...]
    gh = jnp.dot(h, whh_ref[...], preferred_element_type=jnp.float32) + bhh_ref[...]
    r = jax.nn.sigmoid(gi[:, :HID] + gh[:, :HID])
    z = jax.nn.sigmoid(gi[:, HID:2 * HID] + gh[:, HID:2 * HID])
    n = jnp.tanh(gi[:, 2 * HID:] + r * gh[:, 2 * HID:])
    h_new = (1.0 - z) * n + z * h
    h_ref[...] = h_new
    sel = len_ref[...] - 1 == t
    o_ref[...] = jnp.where(sel, h_new, o_ref[...])


def _gru(x, wih_t, whh_t, bih, bhh, lens):
    return pl.pallas_call(
        _gru_kernel,
        grid=(LSEQ,),
        in_specs=[
            pl.BlockSpec((1, BQ, HID), lambda t: (t, 0, 0)),
            pl.BlockSpec((HID, 3 * HID), lambda t: (0, 0)),
            pl.BlockSpec((HID, 3 * HID), lambda t: (0, 0)),
            pl.BlockSpec((1, 3 * HID), lambda t: (0, 0)),
            pl.BlockSpec((1, 3 * HID), lambda t: (0, 0)),
            pl.BlockSpec((BQ, 1), lambda t: (0, 0)),
        ],
        out_specs=pl.BlockSpec((BQ, HID), lambda t: (0, 0)),
        out_shape=jax.ShapeDtypeStruct((BQ, HID), jnp.float32),
        scratch_shapes=[pltpu.VMEM((BQ, HID), jnp.float32)],
    )(x, wih_t, whh_t, bih, bhh, lens)


def _ln_rows(x, g, bb):
    m = jnp.mean(x, axis=1, keepdims=True)
    v = jnp.mean((x - m) ** 2, axis=1, keepdims=True)
    return (x - m) / jnp.sqrt(v + 1e-5) * g + bb


def _scores_kernel(ht_ref, iv_ref, cls_ref, g2_ref, b2_ref, g4_ref, b4_ref,
                   a_ref, s_ref, s1_ref, s2_ref):
    hta = ht_ref[...]
    ht = _ln_rows(hta[:1024], g2_ref[...], b2_ref[...])
    htc = _ln_rows(hta[1024:], g4_ref[...], b4_ref[...])
    dn = (((1,), (1,)), ((), ()))
    s1 = lax.dot_general(ht, iv_ref[...], dn, preferred_element_type=jnp.float32)
    s2 = lax.dot_general(htc, cls_ref[...], dn, preferred_element_type=jnp.float32)
    a1 = jax.nn.sigmoid(a_ref[0, 0])
    a2 = jax.nn.sigmoid(a_ref[0, 1])
    s1_ref[...] = s1
    s2_ref[...] = s2
    s_ref[...] = a1 * s1 + a2 * s2


def _scores(ht_raw, ivp, cls_n, g2, b2, g4, b4, a12):
    nblk = NP // BLK_V
    row = pl.BlockSpec((1, HID), lambda j: (0, 0))
    out_spec = pl.BlockSpec((1024, BLK_V), lambda j: (0, j))
    out_ty = jax.ShapeDtypeStruct((1024, NNODE), jnp.float32)
    return pl.pallas_call(
        _scores_kernel,
        grid=(nblk,),
        in_specs=[
            pl.BlockSpec((BQ, HID), lambda j: (0, 0)),
            pl.BlockSpec((BLK_V, HID), lambda j: (j, 0)),
            pl.BlockSpec((BLK_V, HID), lambda j: (j, 0)),
            row, row, row, row,
            pl.BlockSpec((1, 2), lambda j: (0, 0)),
        ],
        out_specs=[out_spec, out_spec, out_spec],
        out_shape=[out_ty, out_ty, out_ty],
    )(ht_raw, ivp, cls_n, g2, b2, g4, b4, a12)


def kernel(inp_sess, mask_1, mask_inf, lengths, adj_items, item_emb, prob_emb,
           W_cls, W_ih, W_hh, b_ih, b_hh, a1, a2, ln1_g, ln1_b, ln2_g, ln2_b,
           ln3_g, ln3_b, ln4_g, ln4_b):
    f32 = jnp.float32
    x_p = jnp.zeros((NP, HID), f32).at[:NNODE].set(item_emb[1:])
    adj = jnp.zeros((NP, NSAMP), jnp.int32).at[:NNODE].set(
        adj_items[1:].astype(jnp.int32))
    nb_idx = adj.reshape(NP * NSAMP)

    xn = _l2norm_rows(x_p)
    nb1 = _sc_gather(xn, nb_idx).reshape(NP, NSAMP, HID)
    u1 = _routing(x_p, xn, nb1)
    nb2 = _sc_gather(u1, nb_idx).reshape(NP, NSAMP, HID)
    u2 = _routing(u1, u1, nb2)
    ivp = _iv(x_p, u1, u2, ln1_g.reshape(1, HID), ln1_b.reshape(1, HID))

    cf = _cls_table(prob_emb, W_cls.T, ln3_g.reshape(1, HID),
                    ln3_b.reshape(1, HID))

    it_table = jnp.concatenate([jnp.zeros((1, HID), f32), ivp[:NNODE]], axis=0)
    big_table = jnp.concatenate([it_table, cf], axis=0)
    st = inp_sess.astype(jnp.int32).T
    idx2 = jnp.concatenate([st, st + NITEM], axis=1).reshape(-1)
    gathered = _sc_gather(big_table, idx2)
    xseq = gathered.reshape(LSEQ, BQ, HID)

    lens2 = jnp.concatenate([lengths, lengths]).astype(jnp.int32).reshape(BQ, 1)
    ht_raw = _gru(xseq, W_ih.T, W_hh.T, b_ih.reshape(1, -1),
                  b_hh.reshape(1, -1), lens2)

    cls_n = jnp.zeros((NP, HID), f32).at[:NNODE].set(cf[1:])
    a12 = jnp.concatenate([a1, a2]).reshape(1, 2)
    scores, s1, s2 = _scores(ht_raw, ivp, cls_n,
                             ln2_g.reshape(1, HID), ln2_b.reshape(1, HID),
                             ln4_g.reshape(1, HID), ln4_b.reshape(1, HID),
                             a12)
    return scores, s1, s2
